# baseline (device time: 69487 ns/iter reference)
import jax
import jax.numpy as jnp
from jax import lax
from jax.experimental import pallas as pl
from jax.experimental.pallas import tpu as pltpu

N_DEV = 4
E_PER = 8
N_EXP = 32
N_TOK = 2048
D = 512
H = 1024
CAP = 576


def kernel(x, router_W, route_idx, expert_W, shared_W):
    def body(x_ref, rw_ref, idx_ref, ew_ref, sw_ref, out_ref,
             xw_ref, ewb_ref, xcat_ref, y_ref, yin_ref, snd, rcv):
        my = lax.axis_index("i")
        left = lax.rem(my + N_DEV - 1, N_DEV)
        right = lax.rem(my + 1, N_DEV)
        opp = lax.rem(my + 2, N_DEV)

        barrier = pltpu.get_barrier_semaphore()
        for nbr in (left, right, opp):
            pl.semaphore_signal(barrier, inc=1, device_id=(nbr,),
                                device_id_type=pl.DeviceIdType.MESH)

        xf = x_ref[:, :]
        xb = xf.astype(jnp.bfloat16)
        scores = jnp.dot(xb, rw_ref[:, :].astype(jnp.bfloat16),
                         preferred_element_type=jnp.float32)
        probs = jnp.exp(scores - jnp.max(scores, axis=1, keepdims=True))
        probs = probs / jnp.sum(probs, axis=1, keepdims=True)
        idx_all = idx_ref[:, :]
        e_iota = lax.broadcasted_iota(jnp.int32, (N_TOK, N_EXP), 1)
        p_sel = jnp.sum(jnp.where(e_iota == idx_all, probs, 0.0),
                        axis=1, keepdims=True)
        xw_ref[:, :] = (xf * p_sel).astype(jnp.bfloat16)
        ewb_ref[:, :] = ew_ref[:, :, :].astype(jnp.bfloat16).reshape(
            E_PER * D, H)

        chip_of = lax.div(idx_all, E_PER)
        chips_iota = lax.broadcasted_iota(jnp.int32, (N_TOK, N_DEV), 1)
        masks = (chip_of == chips_iota).astype(jnp.float32)
        HB = N_TOK // 2
        tri = (lax.broadcasted_iota(jnp.int32, (HB, HB), 0)
               >= lax.broadcasted_iota(jnp.int32, (HB, HB), 1)
               ).astype(jnp.bfloat16)
        m0 = masks[0:HB, :].astype(jnp.bfloat16)
        m1 = masks[HB:N_TOK, :].astype(jnp.bfloat16)
        p0 = jnp.dot(tri, m0, preferred_element_type=jnp.float32)
        p1 = jnp.dot(tri, m1, preferred_element_type=jnp.float32)
        ranks = (jnp.concatenate([p0, p1 + p0[HB - 1:HB, :]], axis=0)
                 - masks).astype(jnp.int32)
        cap_iota = lax.broadcasted_iota(jnp.int32, (N_TOK, CAP), 1)

        def scatter_mat(r):
            sel = jnp.sum(jnp.where(chips_iota == r, ranks, 0),
                          axis=1, keepdims=True)
            sel = jnp.where(chip_of == r, sel, -1)
            sel = jnp.broadcast_to(sel, (N_TOK, CAP))
            return jnp.where(sel == cap_iota, 1, 0).astype(jnp.bfloat16)

        def tdot(a, b):
            return lax.dot_general(a, b, (((0,), (0,)), ((), ())),
                                   preferred_element_type=jnp.float32)

        S_my = scatter_mat(my)
        xg = tdot(S_my, xw_ref[:, :]).astype(jnp.bfloat16)
        idxg = tdot(S_my, idx_all.astype(jnp.bfloat16))
        for e in range(E_PER):
            ge = my * E_PER + e
            xcat_ref[:, e * D:(e + 1) * D] = jnp.where(
                idxg == ge.astype(jnp.float32), xg,
                jnp.zeros((), jnp.bfloat16))
        y_ref[:, :] = jnp.dot(xcat_ref[:, :], ewb_ref[:, :],
                              preferred_element_type=jnp.float32
                              ).astype(jnp.bfloat16)

        pl.semaphore_wait(barrier, 3)
        started = []
        for sem_idx, dev in ((1, opp), (0, left), (2, right)):
            dd = pltpu.make_async_remote_copy(
                src_ref=y_ref, dst_ref=yin_ref.at[sem_idx],
                send_sem=snd.at[sem_idx], recv_sem=rcv.at[sem_idx],
                device_id=(dev,), device_id_type=pl.DeviceIdType.MESH)
            dd.start()
            started.append(dd)

        swb = sw_ref[:, :].astype(jnp.bfloat16)
        acc = jnp.dot(xb, swb, preferred_element_type=jnp.float32
                      ).astype(jnp.bfloat16)
        acc = acc + jnp.dot(S_my, y_ref[:, :],
                            preferred_element_type=jnp.float32
                            ).astype(jnp.bfloat16)

        for d_off in (1, 2, 3):
            r = lax.rem(my + d_off, N_DEV)
            S_r = scatter_mat(r)
            dd = pltpu.make_async_remote_copy(
                src_ref=y_ref, dst_ref=yin_ref.at[d_off - 1],
                send_sem=snd.at[d_off - 1], recv_sem=rcv.at[d_off - 1],
                device_id=(right,), device_id_type=pl.DeviceIdType.MESH)
            dd.wait_recv()
            acc = acc + jnp.dot(S_r, yin_ref[d_off - 1, :, :],
                                preferred_element_type=jnp.float32
                                ).astype(jnp.bfloat16)

        out_ref[:, :] = acc

        for dd in started:
            dd.wait_send()

    return pl.pallas_call(
        body,
        out_shape=jax.ShapeDtypeStruct((N_TOK, H), jnp.bfloat16),
        in_specs=[pl.BlockSpec(memory_space=pltpu.VMEM)] * 5,
        out_specs=pl.BlockSpec(memory_space=pltpu.VMEM),
        scratch_shapes=[
            pltpu.VMEM((N_TOK, D), jnp.bfloat16),
            pltpu.VMEM((E_PER * D, H), jnp.bfloat16),
            pltpu.VMEM((CAP, E_PER * D), jnp.bfloat16),
            pltpu.VMEM((CAP, H), jnp.bfloat16),
            pltpu.VMEM((N_DEV - 1, CAP, H), jnp.bfloat16),
            pltpu.SemaphoreType.DMA((3,)),
            pltpu.SemaphoreType.DMA((3,)),
        ],
        compiler_params=pltpu.CompilerParams(
            collective_id=0, vmem_limit_bytes=100 * 1024 * 1024),
    )(x, router_W, route_idx, expert_W, shared_W)


# device time: 65354 ns/iter; 1.0632x vs baseline; 1.0632x over previous
import jax
import jax.numpy as jnp
from jax import lax
from jax.experimental import pallas as pl
from jax.experimental.pallas import tpu as pltpu

N_DEV = 4
E_PER = 8
N_EXP = 32
N_TOK = 2048
D = 512
H = 1024
CAP = 576


def kernel(x, router_W, route_idx, expert_W, shared_W):
    def body(x_ref, rw_ref, idx_ref, ew_ref, sw_ref, out_ref,
             xw_ref, ewb_ref, xcat_ref, y_ref, yin_ref, snd, rcv):
        my = lax.axis_index("i")
        left = lax.rem(my + N_DEV - 1, N_DEV)
        right = lax.rem(my + 1, N_DEV)
        opp = lax.rem(my + 2, N_DEV)

        barrier = pltpu.get_barrier_semaphore()
        for nbr in (left, right):
            pl.semaphore_signal(barrier, inc=1, device_id=(nbr,),
                                device_id_type=pl.DeviceIdType.MESH)

        xf = x_ref[:, :]
        xb = xf.astype(jnp.bfloat16)
        scores = jnp.dot(xb, rw_ref[:, :].astype(jnp.bfloat16),
                         preferred_element_type=jnp.float32)
        probs = jnp.exp(scores - jnp.max(scores, axis=1, keepdims=True))
        probs = probs / jnp.sum(probs, axis=1, keepdims=True)
        idx_all = idx_ref[:, :]
        e_iota = lax.broadcasted_iota(jnp.int32, (N_TOK, N_EXP), 1)
        p_sel = jnp.sum(jnp.where(e_iota == idx_all, probs, 0.0),
                        axis=1, keepdims=True)
        xw_ref[:, :] = (xf * p_sel).astype(jnp.bfloat16)
        ewb_ref[:, :] = ew_ref[:, :, :].astype(jnp.bfloat16).reshape(
            E_PER * D, H)

        chip_of = lax.div(idx_all, E_PER)
        chips_iota = lax.broadcasted_iota(jnp.int32, (N_TOK, N_DEV), 1)
        masks = (chip_of == chips_iota).astype(jnp.float32)
        HB = N_TOK // 2
        tri = (lax.broadcasted_iota(jnp.int32, (HB, HB), 0)
               >= lax.broadcasted_iota(jnp.int32, (HB, HB), 1)
               ).astype(jnp.bfloat16)
        m0 = masks[0:HB, :].astype(jnp.bfloat16)
        m1 = masks[HB:N_TOK, :].astype(jnp.bfloat16)
        p0 = jnp.dot(tri, m0, preferred_element_type=jnp.float32)
        p1 = jnp.dot(tri, m1, preferred_element_type=jnp.float32)
        ranks = (jnp.concatenate([p0, p1 + p0[HB - 1:HB, :]], axis=0)
                 - masks).astype(jnp.int32)
        cap_iota = lax.broadcasted_iota(jnp.int32, (N_TOK, CAP), 1)

        def scatter_mat(r):
            sel = jnp.sum(jnp.where(chips_iota == r, ranks, 0),
                          axis=1, keepdims=True)
            sel = jnp.where(chip_of == r, sel, -1)
            sel = jnp.broadcast_to(sel, (N_TOK, CAP))
            return jnp.where(sel == cap_iota, 1, 0).astype(jnp.bfloat16)

        def tdot(a, b):
            return lax.dot_general(a, b, (((0,), (0,)), ((), ())),
                                   preferred_element_type=jnp.float32)

        S_my = scatter_mat(my)
        xg = tdot(S_my, xw_ref[:, :]).astype(jnp.bfloat16)
        idxg = tdot(S_my, idx_all.astype(jnp.bfloat16))
        for e in range(E_PER):
            ge = my * E_PER + e
            xcat_ref[:, e * D:(e + 1) * D] = jnp.where(
                idxg == ge.astype(jnp.float32), xg,
                jnp.zeros((), jnp.bfloat16))
        y_ref[:, :] = jnp.dot(xcat_ref[:, :], ewb_ref[:, :],
                              preferred_element_type=jnp.float32
                              ).astype(jnp.bfloat16)

        pl.semaphore_wait(barrier, 2)
        started = []
        for sem_idx, dev in ((0, right), (1, left)):
            dd = pltpu.make_async_remote_copy(
                src_ref=y_ref, dst_ref=yin_ref.at[sem_idx],
                send_sem=snd.at[sem_idx], recv_sem=rcv.at[sem_idx],
                device_id=(dev,), device_id_type=pl.DeviceIdType.MESH)
            dd.start()
            started.append(dd)

        swb = sw_ref[:, :].astype(jnp.bfloat16)
        acc = jnp.dot(xb, swb, preferred_element_type=jnp.float32
                      ).astype(jnp.bfloat16)
        acc = acc + jnp.dot(S_my, y_ref[:, :],
                            preferred_element_type=jnp.float32
                            ).astype(jnp.bfloat16)

        def recv_wait(slot):
            pltpu.make_async_remote_copy(
                src_ref=y_ref, dst_ref=yin_ref.at[slot],
                send_sem=snd.at[slot], recv_sem=rcv.at[slot],
                device_id=(right,), device_id_type=pl.DeviceIdType.MESH
            ).wait_recv()

        recv_wait(0)
        fwd = pltpu.make_async_remote_copy(
            src_ref=yin_ref.at[0], dst_ref=yin_ref.at[2],
            send_sem=snd.at[2], recv_sem=rcv.at[2],
            device_id=(right,), device_id_type=pl.DeviceIdType.MESH)
        fwd.start()
        started.append(fwd)
        acc = acc + jnp.dot(scatter_mat(left), yin_ref[0, :, :],
                            preferred_element_type=jnp.float32
                            ).astype(jnp.bfloat16)
        recv_wait(1)
        acc = acc + jnp.dot(scatter_mat(right), yin_ref[1, :, :],
                            preferred_element_type=jnp.float32
                            ).astype(jnp.bfloat16)
        recv_wait(2)
        acc = acc + jnp.dot(scatter_mat(opp), yin_ref[2, :, :],
                            preferred_element_type=jnp.float32
                            ).astype(jnp.bfloat16)

        out_ref[:, :] = acc

        for dd in started:
            dd.wait_send()

    return pl.pallas_call(
        body,
        out_shape=jax.ShapeDtypeStruct((N_TOK, H), jnp.bfloat16),
        in_specs=[pl.BlockSpec(memory_space=pltpu.VMEM)] * 5,
        out_specs=pl.BlockSpec(memory_space=pltpu.VMEM),
        scratch_shapes=[
            pltpu.VMEM((N_TOK, D), jnp.bfloat16),
            pltpu.VMEM((E_PER * D, H), jnp.bfloat16),
            pltpu.VMEM((CAP, E_PER * D), jnp.bfloat16),
            pltpu.VMEM((CAP, H), jnp.bfloat16),
            pltpu.VMEM((N_DEV - 1, CAP, H), jnp.bfloat16),
            pltpu.SemaphoreType.DMA((3,)),
            pltpu.SemaphoreType.DMA((3,)),
        ],
        compiler_params=pltpu.CompilerParams(
            collective_id=0, vmem_limit_bytes=100 * 1024 * 1024),
    )(x, router_W, route_idx, expert_W, shared_W)
